# baseline (speedup 1.0000x reference)
import jax
import jax.numpy as jnp
import numpy as np
from jax.experimental import pallas as pl
from jax.experimental.pallas import tpu as pltpu

_NUM_BINS = 10
_FLOAT_EPS = float(np.finfo(np.float16).eps)
_SCALE = float(_NUM_BINS - _FLOAT_EPS)
_ROWS, _COLS = 8192, 4096
_BLOCK_ROWS = 512
_CHUNK = 16
_LN2 = float(np.log(2.0))


def _hist_kernel(p_ref, g_ref, cnt_ref):
    nchunks = _BLOCK_ROWS // _CHUNK

    def body(j, accs):
        r0 = j * _CHUNK
        p = p_ref[pl.ds(r0, _CHUNK), :]
        t = g_ref[pl.ds(r0, _CHUNK), :]
        ind = jnp.floor(jnp.abs(p - t) * _SCALE)
        out = []
        for c in range(1, _NUM_BINS):
            m = jnp.minimum(ind, jnp.float32(c))
            parts = [m[:, 128 * k:128 * (k + 1)] for k in range(_COLS // 128)]
            while len(parts) > 1:
                h = len(parts) // 2
                parts = [parts[i] + parts[h + i] for i in range(h)] + (
                    parts[2 * h:])
            out.append(accs[c - 1] + parts[0])
        return tuple(out)

    init = tuple(jnp.zeros((_CHUNK, 128), jnp.float32)
                 for _ in range(_NUM_BINS - 1))
    accs = jax.lax.fori_loop(0, nchunks, body, init)
    s = [jnp.sum(a).astype(jnp.int32) for a in accs]

    n_blk = jnp.int32(_BLOCK_ROWS * _COLS)
    g_cum = [n_blk] + [s[c - 1] - (s[c - 2] if c > 1 else 0)
                       for c in range(1, _NUM_BINS)]
    cnts = [g_cum[c] - g_cum[c + 1] for c in range(_NUM_BINS - 1)]
    cnts = cnts + [g_cum[_NUM_BINS - 1]]
    cnts_vec = jnp.stack(cnts).reshape(1, _NUM_BINS)

    @pl.when(pl.program_id(0) == 0)
    def _init():
        cnt_ref[...] = cnts_vec

    @pl.when(pl.program_id(0) != 0)
    def _acc():
        cnt_ref[...] = cnt_ref[...] + cnts_vec


def _loss_kernel(cnt_ref, p_ref, g_ref, out_ref):
    n = jnp.float32(_ROWS * _COLS)
    nonempty = jnp.float32(0.0)
    for i in range(_NUM_BINS):
        nonempty = nonempty + (cnt_ref[i] > 0).astype(jnp.float32)
    ws = []
    for i in range(_NUM_BINS):
        gd = jnp.maximum(cnt_ref[i].astype(jnp.float32) * nonempty, _FLOAT_EPS)
        ws.append((-_LN2) * n / gd)

    nchunks = _BLOCK_ROWS // _CHUNK

    def body(j, carry):
        r0 = j * _CHUNK
        p = p_ref[pl.ds(r0, _CHUNK), :]
        t = g_ref[pl.ds(r0, _CHUNK), :]
        x = jnp.abs(p - t) * _SCALE
        w = jnp.full(x.shape, ws[_NUM_BINS - 1], dtype=jnp.float32)
        for i in range(_NUM_BINS - 2, -1, -1):
            w = jnp.where(x < jnp.float32(i + 1), ws[i], w)
        pc = jnp.clip(p, 1e-7, 1.0 - 1e-7)
        q1 = jnp.log2(pc)
        q2 = jnp.log2(1.0 - pc)
        out_ref[pl.ds(r0, _CHUNK), :] = (t * (q1 - q2) + q2) * w
        return carry

    jax.lax.fori_loop(0, nchunks, body, 0)


@jax.jit
def kernel(pconf, gconf):
    n_blocks = _ROWS // _BLOCK_ROWS
    counts = pl.pallas_call(
        _hist_kernel,
        grid=(n_blocks,),
        in_specs=[
            pl.BlockSpec((_BLOCK_ROWS, _COLS), lambda i: (i, 0)),
            pl.BlockSpec((_BLOCK_ROWS, _COLS), lambda i: (i, 0)),
        ],
        out_specs=pl.BlockSpec((1, _NUM_BINS), lambda i: (0, 0)),
        out_shape=jax.ShapeDtypeStruct((1, _NUM_BINS), jnp.int32),
        compiler_params=pltpu.CompilerParams(
            dimension_semantics=("arbitrary",)),
    )(pconf, gconf)

    loss = pl.pallas_call(
        _loss_kernel,
        grid=(n_blocks,),
        in_specs=[
            pl.BlockSpec(memory_space=pltpu.SMEM),
            pl.BlockSpec((_BLOCK_ROWS, _COLS), lambda i: (i, 0)),
            pl.BlockSpec((_BLOCK_ROWS, _COLS), lambda i: (i, 0)),
        ],
        out_specs=pl.BlockSpec((_BLOCK_ROWS, _COLS), lambda i: (i, 0)),
        out_shape=jax.ShapeDtypeStruct((_ROWS, _COLS), jnp.float32),
        compiler_params=pltpu.CompilerParams(
            dimension_semantics=("parallel",)),
    )(counts.reshape(_NUM_BINS), pconf, gconf)
    return loss


# speedup vs baseline: 1.0693x; 1.0693x over previous
import jax
import jax.numpy as jnp
import numpy as np
from jax.experimental import pallas as pl
from jax.experimental.pallas import tpu as pltpu

_NUM_BINS = 10
_FLOAT_EPS = float(np.finfo(np.float16).eps)
_SCALE = float(_NUM_BINS - _FLOAT_EPS)
_ROWS, _COLS = 8192, 4096
_BLOCK_ROWS = 512
_CHUNK = 16
_LN2 = float(np.log(2.0))


def _hist_kernel(p_ref, g_ref, w_ref, *acc_refs):
    nchunks = _BLOCK_ROWS // _CHUNK
    n_blocks = _ROWS // _BLOCK_ROWS

    def body(j, accs):
        r0 = j * _CHUNK
        p = p_ref[pl.ds(r0, _CHUNK), :]
        t = g_ref[pl.ds(r0, _CHUNK), :]
        ind = jnp.floor(jnp.abs(p - t) * _SCALE)
        out = []
        for c in range(1, _NUM_BINS):
            m = jnp.minimum(ind, jnp.float32(c))
            parts = [m[:, 128 * k:128 * (k + 1)] for k in range(_COLS // 128)]
            while len(parts) > 1:
                h = len(parts) // 2
                parts = [parts[i] + parts[h + i] for i in range(h)] + (
                    parts[2 * h:])
            out.append(accs[c - 1] + parts[0])
        return tuple(out)

    init = tuple(jnp.zeros((_CHUNK, 128), jnp.float32)
                 for _ in range(_NUM_BINS - 1))
    accs = jax.lax.fori_loop(0, nchunks, body, init)

    @pl.when(pl.program_id(0) == 0)
    def _first():
        for r, v in zip(acc_refs, accs):
            r[...] = v

    @pl.when(pl.program_id(0) != 0)
    def _rest():
        for r, v in zip(acc_refs, accs):
            r[...] = r[...] + v

    @pl.when(pl.program_id(0) == n_blocks - 1)
    def _finalize():
        s = [jnp.sum(r[...].astype(jnp.int32)) for r in acc_refs]
        n_tot = jnp.int32(_ROWS * _COLS)
        g_cum = [n_tot] + [s[c - 1] - (s[c - 2] if c > 1 else 0)
                           for c in range(1, _NUM_BINS)]
        cnts = [g_cum[c] - g_cum[c + 1] for c in range(_NUM_BINS - 1)]
        cnts = cnts + [g_cum[_NUM_BINS - 1]]
        nonempty = jnp.float32(0.0)
        for c in cnts:
            nonempty = nonempty + (c > 0).astype(jnp.float32)
        n_f = jnp.float32(_ROWS * _COLS)
        ws = []
        for c in cnts:
            gd = jnp.maximum(c.astype(jnp.float32) * nonempty, _FLOAT_EPS)
            ws.append((-_LN2) * n_f / gd)
        w_ref[...] = jnp.stack(ws).reshape(1, _NUM_BINS)


def _loss_kernel(w_ref, p_ref, g_ref, out_ref):
    wsb = [w_ref[i].astype(jnp.bfloat16) for i in range(_NUM_BINS)]
    nchunks = _BLOCK_ROWS // _CHUNK

    def body(j, carry):
        r0 = j * _CHUNK
        p = p_ref[pl.ds(r0, _CHUNK), :]
        t = g_ref[pl.ds(r0, _CHUNK), :]
        idx = jnp.floor(jnp.abs(p - t) * _SCALE).astype(jnp.bfloat16)
        wb = jnp.full(idx.shape, wsb[_NUM_BINS - 1], dtype=jnp.bfloat16)
        for i in range(_NUM_BINS - 2, -1, -1):
            wb = jnp.where(idx < jnp.bfloat16(i + 1), wsb[i], wb)
        w = wb.astype(jnp.float32)
        pc = jnp.clip(p, 1e-7, 1.0 - 1e-7)
        q1 = jnp.log2(pc)
        q2 = jnp.log2(1.0 - pc)
        out_ref[pl.ds(r0, _CHUNK), :] = (t * (q1 - q2) + q2) * w
        return carry

    jax.lax.fori_loop(0, nchunks, body, 0)


@jax.jit
def kernel(pconf, gconf):
    n_blocks = _ROWS // _BLOCK_ROWS
    weights = pl.pallas_call(
        _hist_kernel,
        grid=(n_blocks,),
        in_specs=[
            pl.BlockSpec((_BLOCK_ROWS, _COLS), lambda i: (i, 0)),
            pl.BlockSpec((_BLOCK_ROWS, _COLS), lambda i: (i, 0)),
        ],
        out_specs=pl.BlockSpec((1, _NUM_BINS), lambda i: (0, 0)),
        out_shape=jax.ShapeDtypeStruct((1, _NUM_BINS), jnp.float32),
        scratch_shapes=[pltpu.VMEM((_CHUNK, 128), jnp.float32)
                        for _ in range(_NUM_BINS - 1)],
        compiler_params=pltpu.CompilerParams(
            dimension_semantics=("arbitrary",)),
    )(pconf, gconf)

    loss = pl.pallas_call(
        _loss_kernel,
        grid=(n_blocks,),
        in_specs=[
            pl.BlockSpec(memory_space=pltpu.SMEM),
            pl.BlockSpec((_BLOCK_ROWS, _COLS), lambda i: (i, 0)),
            pl.BlockSpec((_BLOCK_ROWS, _COLS), lambda i: (i, 0)),
        ],
        out_specs=pl.BlockSpec((_BLOCK_ROWS, _COLS), lambda i: (i, 0)),
        out_shape=jax.ShapeDtypeStruct((_ROWS, _COLS), jnp.float32),
        compiler_params=pltpu.CompilerParams(
            dimension_semantics=("parallel",)),
    )(weights.reshape(_NUM_BINS), pconf, gconf)
    return loss
